# Initial kernel scaffold; baseline (speedup 1.0000x reference)
import jax
import jax.numpy as jnp
from jax.experimental import pallas as pl


def kernel(input_ids, sem_node_ids, sememe_positions, sememe_node_idx, table, W):
    raise NotImplementedError("write your pallas kernel here")


# speedup vs baseline: 1.6972x; 1.6972x over previous
import functools

import jax
import jax.numpy as jnp
from jax import lax
from jax.experimental import pallas as pl
from jax.experimental.pallas import tpu as pltpu
from jax.experimental.pallas import tpu_sc as plsc

NC = 2
NS = 16
NW = NC * NS


def _mm_body(nf_ref, w_ref, o_ref):
    o_ref[...] = jnp.tanh(
        jnp.dot(nf_ref[...], w_ref[...], preferred_element_type=jnp.float32))


def _matmul_tanh(nf, w):
    n, d = nf.shape
    grid = 16
    blk = n // grid
    return pl.pallas_call(
        _mm_body,
        grid=(grid,),
        in_specs=[
            pl.BlockSpec((blk, d), lambda i: (i, 0)),
            pl.BlockSpec((d, d), lambda i: (0, 0)),
        ],
        out_specs=pl.BlockSpec((blk, d), lambda i: (i, 0)),
        out_shape=jax.ShapeDtypeStruct((n, d), jnp.float32),
    )(nf, w)


def _worker_id():
    return lax.axis_index("s") * NC + lax.axis_index("c")


def _nodes_body(npw, table, ids, out, idx_v, buf, sem):
    base = _worker_id() * npw
    pltpu.sync_copy(ids.at[pl.ds(base, npw)], idx_v)
    pltpu.async_copy(table.at[idx_v], buf, sem).wait()
    pltpu.sync_copy(buf, out.at[pl.ds(base, npw)])


def _gather_nodes(table, ids):
    nsem = ids.shape[0]
    d = table.shape[1]
    npw = nsem // NW
    mesh = plsc.VectorSubcoreMesh(core_axis_name="c", subcore_axis_name="s")
    return pl.kernel(
        functools.partial(_nodes_body, npw),
        jax.ShapeDtypeStruct((nsem, d), jnp.float32),
        mesh=mesh,
        scratch_types=[
            pltpu.VMEM((npw,), jnp.int32),
            pltpu.VMEM((npw, d), jnp.float32),
            pltpu.SemaphoreType.DMA,
        ],
    )(table, ids)


_CH = 80
_NCH = 20
_ECH = 64


def _main_body(rpw, epw, table, flat_ids, gat, src, dst, out,
               idx_v, sidx_v, didx_v, buf, sem):
    wid = _worker_id()
    rbase = wid * rpw
    for c in range(_NCH):
        off = rbase + c * _CH
        pltpu.sync_copy(flat_ids.at[pl.ds(off, _CH)], idx_v)
        pltpu.async_copy(table.at[idx_v], buf, sem).wait()
        pltpu.sync_copy(buf, out.at[pl.ds(off, _CH)])
    ebase = wid * epw
    for j in range(epw // _ECH):
        off = ebase + j * _ECH
        pltpu.sync_copy(src.at[pl.ds(off, _ECH)], sidx_v)
        pltpu.async_copy(gat.at[sidx_v], buf.at[pl.ds(0, _ECH)], sem).wait()
        pltpu.sync_copy(dst.at[pl.ds(off, _ECH)], didx_v)
        pltpu.async_copy(buf.at[pl.ds(0, _ECH)], out.at[didx_v], sem).wait()


def _main_gather_scatter(table, flat_ids, gat, src, dst):
    rows = flat_ids.shape[0]
    d = table.shape[1]
    rpw = rows // NW
    epw = src.shape[0] // NW
    mesh = plsc.VectorSubcoreMesh(core_axis_name="c", subcore_axis_name="s")
    return pl.kernel(
        functools.partial(_main_body, rpw, epw),
        jax.ShapeDtypeStruct((rows, d), jnp.float32),
        mesh=mesh,
        scratch_types=[
            pltpu.VMEM((_CH,), jnp.int32),
            pltpu.VMEM((_ECH,), jnp.int32),
            pltpu.VMEM((_ECH,), jnp.int32),
            pltpu.VMEM((_CH, d), jnp.float32),
            pltpu.SemaphoreType.DMA,
        ],
    )(table, flat_ids, gat, src, dst)


def kernel(input_ids, sem_node_ids, sememe_positions, sememe_node_idx, table, W):
    b, s = input_ids.shape
    p = sememe_positions.shape[1]
    d = table.shape[1]

    ids_flat = input_ids.reshape(b * s).astype(jnp.int32)
    pos = sememe_positions.astype(jnp.int32)
    nid = sememe_node_idx.astype(jnp.int32)

    eq = pos[:, :, None] == pos[:, None, :]
    parr = jnp.arange(p, dtype=jnp.int32)
    winner = jnp.max(jnp.where(eq, parr[None, None, :], -1), axis=-1)
    src = jnp.take_along_axis(nid, winner, axis=1).reshape(b * p)
    dst = (jnp.arange(b, dtype=jnp.int32)[:, None] * s + pos).reshape(b * p)

    nf = _gather_nodes(table, sem_node_ids.astype(jnp.int32))
    gat = _matmul_tanh(nf, W)
    out_flat = _main_gather_scatter(table, ids_flat, gat, src, dst)
    return out_flat.reshape(b, s, d)


# speedup vs baseline: 1.7567x; 1.0350x over previous
import functools

import jax
import jax.numpy as jnp
from jax import lax
from jax.experimental import pallas as pl
from jax.experimental.pallas import tpu as pltpu
from jax.experimental.pallas import tpu_sc as plsc

NC = 2
NS = 16
NW = NC * NS


def _mm_body(nf_ref, w_ref, o_ref):
    o_ref[...] = jnp.tanh(
        jnp.dot(nf_ref[...], w_ref[...], preferred_element_type=jnp.float32))


def _matmul_tanh(nf, w):
    n, d = nf.shape
    grid = 16
    blk = n // grid
    return pl.pallas_call(
        _mm_body,
        grid=(grid,),
        in_specs=[
            pl.BlockSpec((blk, d), lambda i: (i, 0)),
            pl.BlockSpec((d, d), lambda i: (0, 0)),
        ],
        out_specs=pl.BlockSpec((blk, d), lambda i: (i, 0)),
        out_shape=jax.ShapeDtypeStruct((n, d), jnp.float32),
    )(nf, w)


def _worker_id():
    return lax.axis_index("s") * NC + lax.axis_index("c")


def _nodes_body(npw, table, ids, out, idx_v, buf, sem):
    base = _worker_id() * npw
    pltpu.sync_copy(ids.at[pl.ds(base, npw)], idx_v)
    pltpu.async_copy(table.at[idx_v], buf, sem).wait()
    pltpu.sync_copy(buf, out.at[pl.ds(base, npw)])


def _gather_nodes(table, ids):
    nsem = ids.shape[0]
    d = table.shape[1]
    npw = nsem // NW
    mesh = plsc.VectorSubcoreMesh(core_axis_name="c", subcore_axis_name="s")
    return pl.kernel(
        functools.partial(_nodes_body, npw),
        jax.ShapeDtypeStruct((nsem, d), jnp.float32),
        mesh=mesh,
        scratch_types=[
            pltpu.VMEM((npw,), jnp.int32),
            pltpu.VMEM((npw, d), jnp.float32),
            pltpu.SemaphoreType.DMA,
        ],
    )(table, ids)


_CH = 80
_NCH = 20
_ECH = 64


def _main_body(rpw, epw, table, flat_ids, gat, src2, dst2, out,
               idx_v, sidx_v, didx_v, buf0, buf1, gsem, ssem0, ssem1):
    wid = _worker_id()
    rbase = wid * rpw
    bufs = (buf0, buf1)
    ssems = (ssem0, ssem1)
    pltpu.sync_copy(flat_ids.at[pl.ds(rbase, rpw)], idx_v)
    stores = [None, None]
    for c in range(_NCH):
        k = c % 2
        if stores[k] is not None:
            stores[k].wait()
        pltpu.async_copy(
            table.at[idx_v.at[pl.ds(c * _CH, _CH)]], bufs[k], gsem).wait()
        stores[k] = pltpu.async_copy(
            bufs[k], out.at[pl.ds(rbase + c * _CH, _CH)], ssems[k])
    stores[0].wait()
    stores[1].wait()
    nech = epw // _ECH
    pltpu.sync_copy(src2.at[pl.ds(wid * nech, nech)], sidx_v)
    pltpu.sync_copy(dst2.at[pl.ds(wid * nech, nech)], didx_v)
    for j in range(nech):
        k = j % 2
        pltpu.async_copy(
            gat.at[sidx_v.at[j]], bufs[k].at[pl.ds(0, _ECH)], gsem).wait()
        pltpu.async_copy(
            bufs[k].at[pl.ds(0, _ECH)], out.at[didx_v.at[j]], ssems[k]).wait()


def _main_gather_scatter(table, flat_ids, gat, src2, dst2):
    rows = flat_ids.shape[0]
    d = table.shape[1]
    rpw = rows // NW
    epw = (src2.shape[0] * src2.shape[1]) // NW
    mesh = plsc.VectorSubcoreMesh(core_axis_name="c", subcore_axis_name="s")
    return pl.kernel(
        functools.partial(_main_body, rpw, epw),
        jax.ShapeDtypeStruct((rows, d), jnp.float32),
        mesh=mesh,
        scratch_types=[
            pltpu.VMEM((rpw,), jnp.int32),
            pltpu.VMEM((epw // _ECH, _ECH), jnp.int32),
            pltpu.VMEM((epw // _ECH, _ECH), jnp.int32),
            pltpu.VMEM((_CH, d), jnp.float32),
            pltpu.VMEM((_CH, d), jnp.float32),
            pltpu.SemaphoreType.DMA,
            pltpu.SemaphoreType.DMA,
            pltpu.SemaphoreType.DMA,
        ],
    )(table, flat_ids, gat, src2, dst2)


def kernel(input_ids, sem_node_ids, sememe_positions, sememe_node_idx, table, W):
    b, s = input_ids.shape
    p = sememe_positions.shape[1]
    d = table.shape[1]

    ids_flat = input_ids.reshape(b * s).astype(jnp.int32)
    pos = sememe_positions.astype(jnp.int32)
    nid = sememe_node_idx.astype(jnp.int32)

    eq = pos[:, :, None] == pos[:, None, :]
    parr = jnp.arange(p, dtype=jnp.int32)
    winner = jnp.max(jnp.where(eq, parr[None, None, :], -1), axis=-1)
    src2 = jnp.take_along_axis(nid, winner, axis=1).reshape(b * p // _ECH, _ECH)
    dst2 = (jnp.arange(b, dtype=jnp.int32)[:, None] * s
            + pos).reshape(b * p // _ECH, _ECH)

    nf = _gather_nodes(table, sem_node_ids.astype(jnp.int32))
    gat = _matmul_tanh(nf, W)
    out_flat = _main_gather_scatter(table, ids_flat, gat, src2, dst2)
    return out_flat.reshape(b, s, d)


# speedup vs baseline: 1.8082x; 1.0293x over previous
import functools

import jax
import jax.numpy as jnp
from jax import lax
from jax.experimental import pallas as pl
from jax.experimental.pallas import tpu as pltpu
from jax.experimental.pallas import tpu_sc as plsc

NC = 2
NS = 16
NW = NC * NS


def _worker_id():
    return lax.axis_index("s") * NC + lax.axis_index("c")


def _sc_mesh():
    return plsc.VectorSubcoreMesh(core_axis_name="c", subcore_axis_name="s")


def _mm_body(nf_ref, w_ref, o_ref):
    o_ref[...] = jnp.tanh(
        jnp.dot(nf_ref[...], w_ref[...], preferred_element_type=jnp.float32))


def _matmul_tanh(nf, w):
    n, d = nf.shape
    grid = 16
    blk = n // grid
    return pl.pallas_call(
        _mm_body,
        grid=(grid,),
        in_specs=[
            pl.BlockSpec((blk, d), lambda i: (i, 0)),
            pl.BlockSpec((d, d), lambda i: (0, 0)),
        ],
        out_specs=pl.BlockSpec((blk, d), lambda i: (i, 0)),
        out_shape=jax.ShapeDtypeStruct((n, d), jnp.float32),
    )(nf, w)


def _rows_body(npw, nch, table, ids, out, idx_v, buf, sem):
    base = _worker_id() * npw
    ch = npw // nch
    pltpu.sync_copy(ids.at[pl.ds(base, npw)], idx_v)
    for c in range(nch):
        pltpu.async_copy(
            table.at[idx_v.at[pl.ds(c * ch, ch)]], buf, sem).wait()
        pltpu.sync_copy(buf, out.at[pl.ds(base + c * ch, ch)])


def _gather_rows(table, ids, nch):
    n = ids.shape[0]
    d = table.shape[1]
    npw = n // NW
    return pl.kernel(
        functools.partial(_rows_body, npw, nch),
        jax.ShapeDtypeStruct((n, d), jnp.float32),
        mesh=_sc_mesh(),
        scratch_types=[
            pltpu.VMEM((npw,), jnp.int32),
            pltpu.VMEM((npw // nch, d), jnp.float32),
            pltpu.SemaphoreType.DMA,
        ],
    )(table, ids)


_CH = 80
_NCH = 20


def _main_body(rpw, table, flat_ids, out, idx_v, buf0, buf1, gsem, ssem0, ssem1):
    rbase = _worker_id() * rpw
    bufs = (buf0, buf1)
    ssems = (ssem0, ssem1)
    pltpu.sync_copy(flat_ids.at[pl.ds(rbase, rpw)], idx_v)
    stores = [None, None]
    for c in range(_NCH):
        k = c % 2
        if stores[k] is not None:
            stores[k].wait()
        pltpu.async_copy(
            table.at[idx_v.at[pl.ds(c * _CH, _CH)]], bufs[k], gsem).wait()
        stores[k] = pltpu.async_copy(
            bufs[k], out.at[pl.ds(rbase + c * _CH, _CH)], ssems[k])
    stores[0].wait()
    stores[1].wait()


def _main_gather(table, flat_ids):
    rows = flat_ids.shape[0]
    d = table.shape[1]
    rpw = rows // NW
    return pl.kernel(
        functools.partial(_main_body, rpw),
        jax.ShapeDtypeStruct((rows, d), jnp.float32),
        mesh=_sc_mesh(),
        scratch_types=[
            pltpu.VMEM((rpw,), jnp.int32),
            pltpu.VMEM((_CH, d), jnp.float32),
            pltpu.VMEM((_CH, d), jnp.float32),
            pltpu.SemaphoreType.DMA,
            pltpu.SemaphoreType.DMA,
            pltpu.SemaphoreType.DMA,
        ],
    )(table, flat_ids)


_EB = 8


def _fmt_body(s, p, d, pos_ref, lin_ref, sel_ref, o_ref):
    i = pl.program_id(0)
    for e in range(_EB):
        o_ref[e] = lin_ref[pl.ds(e * s, s), :]
    for e in range(_EB):
        for j in range(p):
            t = pos_ref[i * _EB + e, j]
            o_ref[e, pl.ds(t, 1), :] = sel_ref[pl.ds(e * p + j, 1), :]


def _assemble(lin, sel, pos):
    rows, d = lin.shape
    b, p = pos.shape
    s = rows // b
    grid = b // _EB
    return pl.pallas_call(
        functools.partial(_fmt_body, s, p, d),
        grid=(grid,),
        in_specs=[
            pl.BlockSpec(memory_space=pltpu.SMEM),
            pl.BlockSpec((_EB * s, d), lambda i: (i, 0)),
            pl.BlockSpec((_EB * p, d), lambda i: (i, 0)),
        ],
        out_specs=pl.BlockSpec((_EB, s, d), lambda i: (i, 0, 0)),
        out_shape=jax.ShapeDtypeStruct((b, s, d), jnp.float32),
    )(pos, lin, sel)


def kernel(input_ids, sem_node_ids, sememe_positions, sememe_node_idx, table, W):
    b, s = input_ids.shape
    p = sememe_positions.shape[1]

    ids_flat = input_ids.reshape(b * s).astype(jnp.int32)
    pos = sememe_positions.astype(jnp.int32)
    nid_flat = sememe_node_idx.reshape(b * p).astype(jnp.int32)

    nf = _gather_rows(table, sem_node_ids.astype(jnp.int32), 1)
    gat = _matmul_tanh(nf, W)
    sel = _gather_rows(gat, nid_flat, 2)
    lin = _main_gather(table, ids_flat)
    return _assemble(lin, sel, pos)


# speedup vs baseline: 2.5313x; 1.3999x over previous
import functools

import jax
import jax.numpy as jnp
from jax import lax
from jax.experimental import pallas as pl
from jax.experimental.pallas import tpu as pltpu
from jax.experimental.pallas import tpu_sc as plsc

NC = 2
NS = 16
NW = NC * NS


def _worker_id():
    return lax.axis_index("s") * NC + lax.axis_index("c")


def _sc_mesh():
    return plsc.VectorSubcoreMesh(core_axis_name="c", subcore_axis_name="s")


def _mm_body(nf_ref, w_ref, o_ref):
    o_ref[...] = jnp.tanh(
        jnp.dot(nf_ref[...], w_ref[...], preferred_element_type=jnp.float32))


def _matmul_tanh(nf, w):
    n, d = nf.shape
    grid = 16
    blk = n // grid
    return pl.pallas_call(
        _mm_body,
        grid=(grid,),
        in_specs=[
            pl.BlockSpec((blk, d), lambda i: (i, 0)),
            pl.BlockSpec((d, d), lambda i: (0, 0)),
        ],
        out_specs=pl.BlockSpec((blk, d), lambda i: (i, 0)),
        out_shape=jax.ShapeDtypeStruct((n, d), jnp.float32),
    )(nf, w)


def _rows_body(npw, table, ids, out, idx_v, buf, sem):
    base = _worker_id() * npw
    pltpu.sync_copy(ids.at[pl.ds(base, npw)], idx_v)
    pltpu.async_copy(table.at[idx_v], buf, sem).wait()
    pltpu.sync_copy(buf, out.at[pl.ds(base, npw)])


def _gather_rows(table, ids):
    n = ids.shape[0]
    d = table.shape[1]
    npw = n // NW
    return pl.kernel(
        functools.partial(_rows_body, npw),
        jax.ShapeDtypeStruct((n, d), jnp.float32),
        mesh=_sc_mesh(),
        scratch_types=[
            pltpu.VMEM((npw,), jnp.int32),
            pltpu.VMEM((npw, d), jnp.float32),
            pltpu.SemaphoreType.DMA,
        ],
    )(table, ids)


_GRP = 4


def _main_body(epw, s, p, table, ids2, gat, pos2, nid2, out,
               idx_v, pos_v, nid_v, buf0, buf1, selb0,
               gsem, ssem0, ssem1, bsem):
    wid = _worker_id()
    exbase = wid * epw
    bufs = (buf0, buf1)
    ssems = (ssem0, ssem1)
    pltpu.sync_copy(ids2.at[pl.ds(exbase, epw)], idx_v)
    pltpu.sync_copy(pos2.at[pl.ds(exbase, epw)], pos_v)
    pltpu.sync_copy(nid2.at[pl.ds(exbase * p, epw * p)], nid_v)
    stores = [None, None]
    for e in range(epw):
        k = e % 2
        if stores[k] is not None:
            stores[k].wait()
        pltpu.async_copy(table.at[idx_v.at[e]], bufs[k], gsem).wait()
        stores[k] = pltpu.async_copy(bufs[k], out.at[exbase + e], ssems[k])
    stores[0].wait()
    stores[1].wait()
    scat = []
    for g in range(epw // _GRP):
        if scat:
            for dsc in scat:
                dsc.wait()
            scat = []
        pltpu.async_copy(
            gat.at[nid_v.at[pl.ds(g * _GRP * p, _GRP * p)]], selb0, gsem
        ).wait()
        for e in range(_GRP):
            ex = g * _GRP + e
            scat.append(pltpu.async_copy(
                selb0.at[pl.ds(e * p, p)],
                out.at[exbase + ex].at[pos_v.at[ex]],
                bsem))
    for dsc in scat:
        dsc.wait()


def _main_gather_scatter(table, ids2, gat, pos2, nid2):
    b, s = ids2.shape
    p = pos2.shape[1]
    d = table.shape[1]
    epw = b // NW
    return pl.kernel(
        functools.partial(_main_body, epw, s, p),
        jax.ShapeDtypeStruct((b, s, d), jnp.float32),
        mesh=_sc_mesh(),
        scratch_types=[
            pltpu.VMEM((b // NW, s), jnp.int32),
            pltpu.VMEM((b // NW, p), jnp.int32),
            pltpu.VMEM((b // NW * p,), jnp.int32),
            pltpu.VMEM((s, d), jnp.float32),
            pltpu.VMEM((s, d), jnp.float32),
            pltpu.VMEM((_GRP * p, d), jnp.float32),
            pltpu.SemaphoreType.DMA,
            pltpu.SemaphoreType.DMA,
            pltpu.SemaphoreType.DMA,
            pltpu.SemaphoreType.DMA,
        ],
    )(table, ids2, gat, pos2, nid2)


def kernel(input_ids, sem_node_ids, sememe_positions, sememe_node_idx, table, W):
    b, s = input_ids.shape
    p = sememe_positions.shape[1]

    ids2 = input_ids.astype(jnp.int32)
    pos = sememe_positions.astype(jnp.int32)
    nid = sememe_node_idx.astype(jnp.int32)

    eq = pos[:, :, None] == pos[:, None, :]
    parr = jnp.arange(p, dtype=jnp.int32)
    winner = jnp.max(jnp.where(eq, parr[None, None, :], -1), axis=-1)
    nid_w = jnp.take_along_axis(nid, winner, axis=1).reshape(b * p)

    nf = _gather_rows(table, sem_node_ids.astype(jnp.int32))
    gat = _matmul_tanh(nf, W)
    return _main_gather_scatter(table, ids2, gat, pos, nid_w)
